# baseline (speedup 1.0000x reference)
import functools

import jax
import jax.numpy as jnp
from jax import lax
from jax.experimental import pallas as pl
from jax.experimental.pallas import tpu as pltpu
from jax.experimental.pallas import tpu_sc as plsc

_N_E = 8192
_E_DIM = 32
_N_TOKENS = 32768

_T = 512
_G = _N_TOKENS // _T

_NC = 2
_NS = 16
_NW = _NC * _NS
_B_PER_W = _N_TOKENS // _NW
_CHUNK = 128
_NCHUNK = _B_PER_W // _CHUNK


_C = 128
_S = 128


def _dist_argmin_body(x_ref, w_ref, idx_ref, wtn_ref, wsq_ref, xwn_ref, xsq_ref):
    @pl.when(pl.program_id(0) == 0)
    def _init():
        wtn0 = jnp.transpose(w_ref[...]) * jnp.float32(-2.0)
        wtn_ref[...] = wtn0
        wsq_ref[...] = 0.25 * jnp.sum(wtn0 * wtn0, axis=0, keepdims=True)

    xb = x_ref[...]
    wtn = wtn_ref[...]
    xwn_ref[...] = jnp.dot(xb, wtn, preferred_element_type=jnp.float32)
    xsq_ref[...] = jnp.sum(xb * xb, axis=1, keepdims=True)
    lane = lax.broadcasted_iota(jnp.int32, (_S, _C), 1).astype(jnp.float32)

    def strip(s, carry):
        xsq_s = xsq_ref[pl.ds(s * _S, _S), :]
        wsq = wsq_ref[...]
        run_min = jnp.zeros((_S, _C), jnp.float32)
        run_c = jnp.zeros((_S, _C), jnp.float32)
        for c in range(_N_E // _C):
            xwn_sc = xwn_ref[pl.ds(s * _S, _S), c * _C:(c + 1) * _C]
            d_c = (xsq_s + wsq[:, c * _C:(c + 1) * _C]) + xwn_sc
            if c == 0:
                run_min = d_c
            else:
                mask = d_c < run_min
                run_min = jnp.where(mask, d_c, run_min)
                run_c = jnp.where(mask, jnp.float32(c), run_c)
        jf = run_c * jnp.float32(_C) + lane
        dmin = jnp.min(run_min, axis=1, keepdims=True)
        cand = jnp.where(run_min == dmin, jf, jnp.float32(_N_E))
        idx_ref[pl.ds(s * _S, _S)] = jnp.min(cand, axis=1).astype(jnp.int32)
        return carry

    lax.fori_loop(0, _T // _S, strip, 0)


def _argmin_tc(x, wt):
    return pl.pallas_call(
        _dist_argmin_body,
        grid=(_G,),
        in_specs=[
            pl.BlockSpec((_T, _E_DIM), lambda i: (i, 0)),
            pl.BlockSpec((_N_E, _E_DIM), lambda i: (0, 0)),
        ],
        out_specs=pl.BlockSpec((_T,), lambda i: (i,)),
        out_shape=jax.ShapeDtypeStruct((_N_TOKENS,), jnp.int32),
        scratch_shapes=[pltpu.VMEM((_E_DIM, _N_E), jnp.float32),
                        pltpu.VMEM((1, _N_E), jnp.float32),
                        pltpu.VMEM((_T, _N_E), jnp.float32),
                        pltpu.VMEM((_T, 1), jnp.float32)],
    )(x, wt)


def _sc_gather_body(table_hbm, idx_hbm, out_hbm, idx_v, rows_v, sem):
    wid = lax.axis_index("s") * _NC + lax.axis_index("c")
    base = wid * _B_PER_W
    for j in range(_NCHUNK):
        pltpu.sync_copy(idx_hbm.at[pl.ds(base + j * _CHUNK, _CHUNK)],
                        idx_v.at[j])
    copies = []
    for j in range(_NCHUNK):
        copies.append(pltpu.async_copy(
            table_hbm.at[idx_v.at[j]],
            rows_v.at[pl.ds(j * _CHUNK, _CHUNK)], sem))
    for c in copies:
        c.wait()
    pltpu.sync_copy(rows_v, out_hbm.at[pl.ds(base, _B_PER_W)])


@functools.cache
def _sc_gather():
    return pl.kernel(
        _sc_gather_body,
        out_type=jax.ShapeDtypeStruct((_N_TOKENS, _E_DIM), jnp.float32),
        mesh=plsc.VectorSubcoreMesh(core_axis_name="c", subcore_axis_name="s"),
        scratch_types=[
            pltpu.VMEM((_NCHUNK, _CHUNK), jnp.int32),
            pltpu.VMEM((_B_PER_W, _E_DIM), jnp.float32),
            pltpu.SemaphoreType.DMA,
        ],
        compiler_params=pltpu.CompilerParams(use_tc_tiling_on_sc=False),
    )


def kernel(x, W):
    min_indexes = _argmin_tc(x, W)
    z_q = _sc_gather()(W, min_indexes)
    return (z_q, min_indexes)


# speedup vs baseline: 1.6885x; 1.6885x over previous
import functools

import jax
import jax.numpy as jnp
from jax import lax
from jax.experimental import pallas as pl
from jax.experimental.pallas import tpu as pltpu
from jax.experimental.pallas import tpu_sc as plsc

_N_E = 8192
_E_DIM = 32
_N_TOKENS = 32768

_T = 512
_G = _N_TOKENS // _T

_NC = 2
_NS = 16
_NW = _NC * _NS
_B_PER_W = _N_TOKENS // _NW
_CHUNK = 128
_NCHUNK = _B_PER_W // _CHUNK


_C = 128


def _dist_argmin_body(x_ref, w_ref, idx_ref, wtn_ref, wsq_ref):
    @pl.when(pl.program_id(0) == 0)
    def _init():
        wtn0 = jnp.transpose(w_ref[...]) * jnp.float32(-2.0)
        wtn_ref[...] = wtn0
        wsq_ref[...] = 0.25 * jnp.sum(wtn0 * wtn0, axis=0, keepdims=True)

    xb = x_ref[...]
    wtn = wtn_ref[...]
    xwn = jnp.dot(xb, wtn, preferred_element_type=jnp.float32)
    xsq = jnp.sum(xb * xb, axis=1, keepdims=True)
    wsq = wsq_ref[...]
    run_min = jnp.zeros((_T, _C), jnp.float32)
    run_c = jnp.zeros((_T, _C), jnp.float32)
    for c in range(_N_E // _C):
        d_c = (xsq + wsq[:, c * _C:(c + 1) * _C]) + xwn[:, c * _C:(c + 1) * _C]
        if c == 0:
            run_min = d_c
        else:
            mask = d_c < run_min
            run_min = jnp.where(mask, d_c, run_min)
            run_c = jnp.where(mask, jnp.float32(c), run_c)
    lane = lax.broadcasted_iota(jnp.int32, (_T, _C), 1).astype(jnp.float32)
    jf = run_c * jnp.float32(_C) + lane
    dmin = jnp.min(run_min, axis=1, keepdims=True)
    cand = jnp.where(run_min == dmin, jf, jnp.float32(_N_E))
    idx_ref[...] = jnp.min(cand, axis=1).astype(jnp.int32)


def _argmin_tc(x, wt):
    return pl.pallas_call(
        _dist_argmin_body,
        grid=(_G,),
        in_specs=[
            pl.BlockSpec((_T, _E_DIM), lambda i: (i, 0)),
            pl.BlockSpec((_N_E, _E_DIM), lambda i: (0, 0)),
        ],
        out_specs=pl.BlockSpec((_T,), lambda i: (i,)),
        out_shape=jax.ShapeDtypeStruct((_N_TOKENS,), jnp.int32),
        scratch_shapes=[pltpu.VMEM((_E_DIM, _N_E), jnp.float32),
                        pltpu.VMEM((1, _N_E), jnp.float32)],
    )(x, wt)


def _sc_gather_body(table_hbm, idx_hbm, out_hbm, idx_v, rows_v, sem):
    wid = lax.axis_index("s") * _NC + lax.axis_index("c")
    base = wid * _B_PER_W
    for j in range(_NCHUNK):
        pltpu.sync_copy(idx_hbm.at[pl.ds(base + j * _CHUNK, _CHUNK)],
                        idx_v.at[j])
    copies = []
    for j in range(_NCHUNK):
        copies.append(pltpu.async_copy(
            table_hbm.at[idx_v.at[j]],
            rows_v.at[pl.ds(j * _CHUNK, _CHUNK)], sem))
    for c in copies:
        c.wait()
    pltpu.sync_copy(rows_v, out_hbm.at[pl.ds(base, _B_PER_W)])


@functools.cache
def _sc_gather():
    return pl.kernel(
        _sc_gather_body,
        out_type=jax.ShapeDtypeStruct((_N_TOKENS, _E_DIM), jnp.float32),
        mesh=plsc.VectorSubcoreMesh(core_axis_name="c", subcore_axis_name="s"),
        scratch_types=[
            pltpu.VMEM((_NCHUNK, _CHUNK), jnp.int32),
            pltpu.VMEM((_B_PER_W, _E_DIM), jnp.float32),
            pltpu.SemaphoreType.DMA,
        ],
        compiler_params=pltpu.CompilerParams(use_tc_tiling_on_sc=False),
    )


def kernel(x, W):
    min_indexes = _argmin_tc(x, W)
    z_q = _sc_gather()(W, min_indexes)
    return (z_q, min_indexes)
